# Initial kernel scaffold; baseline (speedup 1.0000x reference)
import jax
import jax.numpy as jnp
from jax.experimental import pallas as pl


def kernel(t, l, emb_w, task_w, W1, b1, W2, b2):
    raise NotImplementedError("write your pallas kernel here")


# speedup vs baseline: 5.0811x; 5.0811x over previous
import functools

import jax
import jax.numpy as jnp
from jax import lax
from jax.experimental import pallas as pl
from jax.experimental.pallas import tpu as pltpu
from jax.experimental.pallas import tpu_sc as plsc

VOCAB = 65536
MD = 512
HALF = MD // 2
BL = 16384 * 20

NC, NS = 2, 16
NW = NC * NS
PER_W = BL // NW
CHUNK = 128
NCHUNK = PER_W // CHUNK
PAD = 2 * CHUNK

RM = 2048
RF = 1024

_SQRT_HALF = 0.7071067811865476


def _pre_body(emb_ref, w1ct_ref, w1tt_ref, task_ref, b1_ref, g_ref, t2_ref):
    g_ref[...] = jnp.dot(emb_ref[...], w1ct_ref[...],
                         preferred_element_type=jnp.float32)

    @pl.when(pl.program_id(0) == 0)
    def _():
        t2_ref[...] = jnp.dot(task_ref[...], w1tt_ref[...],
                              preferred_element_type=jnp.float32) + b1_ref[...]


_pre_call = pl.pallas_call(
    _pre_body,
    grid=(VOCAB // RM,),
    in_specs=[
        pl.BlockSpec((RM, MD), lambda i: (i, 0)),
        pl.BlockSpec((MD, HALF), lambda i: (0, 0)),
        pl.BlockSpec((MD, HALF), lambda i: (0, 0)),
        pl.BlockSpec((2, MD), lambda i: (0, 0)),
        pl.BlockSpec((1, HALF), lambda i: (0, 0)),
    ],
    out_specs=[
        pl.BlockSpec((RM, HALF), lambda i: (i, 0)),
        pl.BlockSpec((2, HALF), lambda i: (0, 0)),
    ],
    out_shape=[
        jax.ShapeDtypeStruct((VOCAB, HALF), jnp.float32),
        jax.ShapeDtypeStruct((2, HALF), jnp.float32),
    ],
)


def _gather_body(tbl, idx, out, idx0, idx1, rows0, rows1, sem0, sem1):
    wid = lax.axis_index("s") * NC + lax.axis_index("c")
    base = wid * PER_W

    def fire(idx_ref, rows_ref, sem, chunk):
        off = base + chunk * CHUNK
        pltpu.sync_copy(idx.at[pl.ds(off, CHUNK)], idx_ref)
        pltpu.make_async_copy(tbl.at[idx_ref], rows_ref, sem).start()

    fire(idx0, rows0, sem0, 0)
    fire(idx1, rows1, sem1, 1)

    def body(i, carry):
        j = 2 * i
        pltpu.make_async_copy(tbl.at[idx0], rows0, sem0).wait()
        pltpu.sync_copy(rows0, out.at[pl.ds(base + j * CHUNK, CHUNK)])
        fire(idx0, rows0, sem0, j + 2)
        pltpu.make_async_copy(tbl.at[idx1], rows1, sem1).wait()
        pltpu.sync_copy(rows1, out.at[pl.ds(base + (j + 1) * CHUNK, CHUNK)])
        fire(idx1, rows1, sem1, j + 3)
        return carry

    lax.fori_loop(0, NCHUNK // 2, body, 0)
    pltpu.make_async_copy(tbl.at[idx0], rows0, sem0).wait()
    pltpu.make_async_copy(tbl.at[idx1], rows1, sem1).wait()


def _make_gather_call():
    return functools.partial(
        pl.kernel,
        mesh=plsc.VectorSubcoreMesh(core_axis_name="c", subcore_axis_name="s",
                                    num_cores=NC, num_subcores=NS),
        out_type=jax.ShapeDtypeStruct((BL, HALF), jnp.float32),
        scratch_types=[
            pltpu.VMEM((CHUNK,), jnp.int32),
            pltpu.VMEM((CHUNK,), jnp.int32),
            pltpu.VMEM((CHUNK, HALF), jnp.float32),
            pltpu.VMEM((CHUNK, HALF), jnp.float32),
            pltpu.SemaphoreType.DMA,
            pltpu.SemaphoreType.DMA,
        ],
    )(_gather_body)


def _ffn_body(g_ref, lf_ref, t2_ref, w2t_ref, b2_ref, o_ref):
    t0 = t2_ref[0:1, :]
    td = t2_ref[1:2, :] - t0
    h = g_ref[...] + t0 + lf_ref[...] * td
    h = 0.5 * h * (1.0 + lax.erf(h * _SQRT_HALF))
    o_ref[...] = jnp.dot(h, w2t_ref[...],
                         preferred_element_type=jnp.float32) + b2_ref[...]


_ffn_call = pl.pallas_call(
    _ffn_body,
    grid=(BL // RF,),
    in_specs=[
        pl.BlockSpec((RF, HALF), lambda i: (i, 0)),
        pl.BlockSpec((RF, 1), lambda i: (i, 0)),
        pl.BlockSpec((2, HALF), lambda i: (0, 0)),
        pl.BlockSpec((HALF, MD), lambda i: (0, 0)),
        pl.BlockSpec((1, MD), lambda i: (0, 0)),
    ],
    out_specs=pl.BlockSpec((RF, MD), lambda i: (i, 0)),
    out_shape=jax.ShapeDtypeStruct((BL, MD), jnp.float32),
)


def kernel(t, l, emb_w, task_w, W1, b1, W2, b2):
    B, L = t.shape
    tf = jnp.concatenate(
        [t.reshape(-1).astype(jnp.int32), jnp.zeros((PAD,), jnp.int32)])
    lf = l.reshape(-1, 1).astype(jnp.float32)
    w1ct = W1[:, :MD].T
    w1tt = W1[:, MD:].T
    w2t = W2.T

    g_tbl, t2 = _pre_call(emb_w, w1ct, w1tt, task_w, b1.reshape(1, HALF))
    g = _make_gather_call()(g_tbl, tf)
    o = _ffn_call(g, lf, t2, w2t, b2.reshape(1, MD))
    return o.reshape(B, L, MD)


# speedup vs baseline: 12.5112x; 2.4623x over previous
import functools

import jax
import jax.numpy as jnp
from jax import lax
from jax.experimental import pallas as pl
from jax.experimental.pallas import tpu as pltpu
from jax.experimental.pallas import tpu_sc as plsc

VOCAB = 65536
MD = 512
HALF = MD // 2
BL = 16384 * 20

NC, NS = 2, 16
NW = NC * NS
PER_W = BL // NW
CHUNK = 128
NCHUNK = PER_W // CHUNK
PAD = 2 * CHUNK

RM = 2048
NM = VOCAB // RM
RF = 1024

_SQRT_HALF = 0.7071067811865476


def _pre_body(emb_ref, w1ct_ref, w1tt_ref, task_ref, b1_ref, g_ref):
    t2 = jnp.dot(task_ref[...], w1tt_ref[...],
                 preferred_element_type=jnp.float32) + b1_ref[...]
    e = pl.program_id(0) // NM
    t2_row = jnp.where(e == 0, t2[0:1, :], t2[1:2, :])
    g_ref[...] = jnp.dot(emb_ref[...], w1ct_ref[...],
                         preferred_element_type=jnp.float32) + t2_row


_pre_call = pl.pallas_call(
    _pre_body,
    grid=(2 * NM,),
    in_specs=[
        pl.BlockSpec((RM, MD), lambda i: (i % NM, 0)),
        pl.BlockSpec((MD, HALF), lambda i: (0, 0)),
        pl.BlockSpec((MD, HALF), lambda i: (0, 0)),
        pl.BlockSpec((2, MD), lambda i: (0, 0)),
        pl.BlockSpec((1, HALF), lambda i: (0, 0)),
    ],
    out_specs=pl.BlockSpec((RM, HALF), lambda i: (i, 0)),
    out_shape=jax.ShapeDtypeStruct((2 * VOCAB, HALF), jnp.float32),
)


def _gather_body(tbl, idx, out, idx0, idx1, rows0, rows1, sem0, sem1):
    wid = lax.axis_index("s") * NC + lax.axis_index("c")
    base = wid * PER_W

    def fire(idx_ref, rows_ref, sem, chunk):
        off = base + chunk * CHUNK
        pltpu.sync_copy(idx.at[pl.ds(off, CHUNK)], idx_ref)
        pltpu.make_async_copy(tbl.at[idx_ref], rows_ref, sem).start()

    fire(idx0, rows0, sem0, 0)
    fire(idx1, rows1, sem1, 1)

    def body(i, carry):
        j = 2 * i
        pltpu.make_async_copy(tbl.at[idx0], rows0, sem0).wait()
        pltpu.sync_copy(rows0, out.at[pl.ds(base + j * CHUNK, CHUNK)])
        fire(idx0, rows0, sem0, j + 2)
        pltpu.make_async_copy(tbl.at[idx1], rows1, sem1).wait()
        pltpu.sync_copy(rows1, out.at[pl.ds(base + (j + 1) * CHUNK, CHUNK)])
        fire(idx1, rows1, sem1, j + 3)
        return carry

    lax.fori_loop(0, NCHUNK // 2, body, 0)
    pltpu.make_async_copy(tbl.at[idx0], rows0, sem0).wait()
    pltpu.make_async_copy(tbl.at[idx1], rows1, sem1).wait()


def _make_gather_call():
    return functools.partial(
        pl.kernel,
        mesh=plsc.VectorSubcoreMesh(core_axis_name="c", subcore_axis_name="s",
                                    num_cores=NC, num_subcores=NS),
        out_type=jax.ShapeDtypeStruct((BL, HALF), jnp.float32),
        scratch_types=[
            pltpu.VMEM((CHUNK,), jnp.int32),
            pltpu.VMEM((CHUNK,), jnp.int32),
            pltpu.VMEM((CHUNK, HALF), jnp.float32),
            pltpu.VMEM((CHUNK, HALF), jnp.float32),
            pltpu.SemaphoreType.DMA,
            pltpu.SemaphoreType.DMA,
        ],
    )(_gather_body)


def _ffn_body(g_ref, w2t_ref, b2_ref, o_ref):
    h = g_ref[...]
    h = 0.5 * h * (1.0 + lax.erf(h * _SQRT_HALF))
    o_ref[...] = jnp.dot(h, w2t_ref[...],
                         preferred_element_type=jnp.float32) + b2_ref[...]


_ffn_call = pl.pallas_call(
    _ffn_body,
    grid=(BL // RF,),
    in_specs=[
        pl.BlockSpec((RF, HALF), lambda i: (i, 0)),
        pl.BlockSpec((HALF, MD), lambda i: (0, 0)),
        pl.BlockSpec((1, MD), lambda i: (0, 0)),
    ],
    out_specs=pl.BlockSpec((RF, MD), lambda i: (i, 0)),
    out_shape=jax.ShapeDtypeStruct((BL, MD), jnp.float32),
)


def kernel(t, l, emb_w, task_w, W1, b1, W2, b2):
    B, L = t.shape
    tl = (t + VOCAB * l).T.reshape(-1).astype(jnp.int32)
    tf = jnp.concatenate([tl, jnp.zeros((PAD,), jnp.int32)])
    w1ct = W1[:, :MD].T
    w1tt = W1[:, MD:].T
    w2t = W2.T

    g_tbl = _pre_call(emb_w, w1ct, w1tt, task_w, b1.reshape(1, HALF))
    g = _make_gather_call()(g_tbl, tf)
    o = _ffn_call(g, w2t, b2.reshape(1, MD))
    return o.reshape(L, B, MD).transpose(1, 0, 2)
